# baseline (speedup 1.0000x reference)
import jax
import jax.numpy as jnp
from jax import lax
from jax.experimental import pallas as pl
from jax.experimental.pallas import tpu as pltpu
from jax.experimental.pallas import tpu_sc as plsc

VOCAB = 1000000
EMB = 64
HID = 128
B = 4096
L = 200

NC = 2
NS = 16
NW = NC * NS
ROWS_TOTAL = 2 * B
ROWS_PER_W = ROWS_TOTAL // NW
FULL_TILES = VOCAB // 128
MAIN_T = FULL_TILES // NW
EXTRA = FULL_TILES - MAIN_T * NW
CHUNKS = ((0, 128), (128, 72))


def _transpose_tile(in_v, out_v, ncols):
    iota = lax.iota(jnp.int32, 16)

    def col_body(v, carry):
        cols = jnp.full((16,), v, jnp.int32)
        for e in range(4):
            rows = iota + e * 16
            vals = plsc.load_gather(in_v, [rows, cols])
            out_v[pl.ds(v * EMB + e * 16, 16)] = vals
        return carry

    lax.fori_loop(0, ncols, col_body, 0)


def _repack_body(tt_hbm, tail_hbm, out_hbm, in_v, out_flat, sem_in, sem_out):
    wid = lax.axis_index("s") * NC + lax.axis_index("c")
    t0 = wid * MAIN_T

    def tile_body(t, carry):
        col0 = pl.multiple_of((t0 + t) * 128, 128)
        pltpu.async_copy(tt_hbm.at[:, pl.ds(col0, 128)], in_v, sem_in).wait()
        _transpose_tile(in_v, out_flat, 128)
        pltpu.async_copy(out_flat, out_hbm.at[pl.ds(col0 * EMB, 128 * EMB)],
                         sem_out).wait()
        return carry

    lax.fori_loop(0, MAIN_T, tile_body, 0)

    for k in range(EXTRA):
        @pl.when(wid == k)
        def _():
            col0 = (FULL_TILES - EXTRA + k) * 128
            pltpu.async_copy(tt_hbm.at[:, pl.ds(col0, 128)], in_v,
                             sem_in).wait()
            _transpose_tile(in_v, out_flat, 128)
            pltpu.async_copy(out_flat,
                             out_hbm.at[pl.ds(col0 * EMB, 128 * EMB)],
                             sem_out).wait()

    @pl.when(wid == NW - 1)
    def _():
        col0 = FULL_TILES * 128
        ntail = VOCAB - col0
        pltpu.async_copy(tail_hbm, in_v, sem_in).wait()
        _transpose_tile(in_v, out_flat, ntail)
        pltpu.async_copy(out_flat.at[pl.ds(0, ntail * EMB)],
                         out_hbm.at[pl.ds(col0 * EMB, ntail * EMB)],
                         sem_out).wait()


def _gather_body(idx_hbm, table_hbm, out_hbm, idx_v, buf, out_v, sem):
    wid = lax.axis_index("s") * NC + lax.axis_index("c")
    base = wid * ROWS_PER_W
    pltpu.sync_copy(idx_hbm.at[pl.ds(base * L, ROWS_PER_W * L)], idx_v)

    def accum_body(r, accs):
        new = list(accs)
        for r2 in range(4):
            for c in range(4):
                new[c] = new[c] + buf[r * 4 + r2, pl.ds(c * 16, 16)]
        return tuple(new)

    def row_body(i, carry):
        off = pl.multiple_of(i * L, 8)
        copies = [
            pltpu.async_copy(
                table_hbm.at[idx_v.at[pl.ds(off + c0, n)]],
                buf.at[pl.ds(c0, n)],
                sem,
            )
            for (c0, n) in CHUNKS
        ]
        for c in copies:
            c.wait()
        zeros = tuple(jnp.zeros((16,), jnp.float32) for _ in range(4))
        accs = lax.fori_loop(0, L // 4, accum_body, zeros)
        for c in range(4):
            out_v[i, pl.ds(c * 16, 16)] = accs[c] * (1.0 / L)
        return carry

    lax.fori_loop(0, ROWS_PER_W, row_body, 0)
    pltpu.sync_copy(out_v, out_hbm.at[pl.ds(base, ROWS_PER_W)])


@jax.jit
def _sc_pooled(idx_flat, table):
    mesh = plsc.VectorSubcoreMesh(
        core_axis_name="c", subcore_axis_name="s", num_cores=NC, num_subcores=NS
    )
    tail = jnp.pad(table.T[:, FULL_TILES * 128:], ((0, 0), (0, 64)))
    flat = pl.kernel(
        _repack_body,
        out_type=jax.ShapeDtypeStruct((VOCAB * EMB,), jnp.float32),
        mesh=mesh,
        scratch_types=[
            pltpu.VMEM((EMB, 128), jnp.float32),
            pltpu.VMEM((128 * EMB,), jnp.float32),
            pltpu.SemaphoreType.DMA,
            pltpu.SemaphoreType.DMA,
        ],
        compiler_params=pltpu.CompilerParams(
            use_tc_tiling_on_sc=True, needs_layout_passes=False
        ),
        name="sc_repack",
    )(table.T, tail)
    return pl.kernel(
        _gather_body,
        out_type=jax.ShapeDtypeStruct((ROWS_TOTAL, EMB), jnp.float32),
        mesh=mesh,
        scratch_types=[
            pltpu.VMEM((ROWS_PER_W * L,), jnp.int32),
            pltpu.VMEM((L, EMB), jnp.float32),
            pltpu.VMEM((ROWS_PER_W, EMB), jnp.float32),
            pltpu.SemaphoreType.DMA,
        ],
        compiler_params=pltpu.CompilerParams(use_tc_tiling_on_sc=False),
        name="sc_gather_mean",
    )(idx_flat, flat.reshape(VOCAB, EMB))


def _mlp_body(xc_ref, xs_ref, w1_ref, b1_ref, w2_ref, b2_ref, o_ref):
    w1 = w1_ref[...]
    b1 = b1_ref[...]
    w2 = w2_ref[...]
    b2 = b2_ref[...]
    hc = jax.nn.relu(
        jnp.dot(xc_ref[...], w1, preferred_element_type=jnp.float32) + b1
    )
    hs = jax.nn.relu(
        jnp.dot(xs_ref[...], w1, preferred_element_type=jnp.float32) + b1
    )
    hc = jax.nn.relu(jnp.dot(hc, w2, preferred_element_type=jnp.float32) + b2)
    hs = jax.nn.relu(jnp.dot(hs, w2, preferred_element_type=jnp.float32) + b2)
    d = hc - hs
    o_ref[...] = jnp.sqrt(jnp.sum(d * d, axis=1))


@jax.jit
def _mlp_distance(xc, xs, w1t, b1, w2t, b2):
    return pl.pallas_call(
        _mlp_body,
        out_shape=jax.ShapeDtypeStruct((B,), jnp.float32),
    )(xc, xs, w1t, b1, w2t, b2)


def kernel(correct_sql, student_sql, table, fc_w, fc_b, out_w, out_b):
    idx_flat = jnp.concatenate(
        [correct_sql.astype(jnp.int32), student_sql.astype(jnp.int32)], axis=0
    ).reshape(-1)
    pooled = _sc_pooled(idx_flat, table)
    xc = pooled[:B]
    xs = pooled[B:]
    return _mlp_distance(
        xc, xs, fc_w.T, fc_b[None, :], out_w.T, out_b[None, :]
    )


# speedup vs baseline: 1.1771x; 1.1771x over previous
import jax
import jax.numpy as jnp
from jax import lax
from jax.experimental import pallas as pl
from jax.experimental.pallas import tpu as pltpu
from jax.experimental.pallas import tpu_sc as plsc

VOCAB = 1000000
EMB = 64
HID = 128
B = 4096
L = 200

NC = 2
NS = 16
NW = NC * NS
ROWS_TOTAL = 2 * B
ROWS_PER_W = ROWS_TOTAL // NW
FULL_TILES = VOCAB // 128
MAIN_T = FULL_TILES // NW
EXTRA = FULL_TILES - MAIN_T * NW
CHUNKS = ((0, 128), (128, 72))


def _transpose_tile(in_v, out_v, ncols):
    iota = lax.iota(jnp.int32, 16)
    rows = [iota + e * 16 for e in range(4)]

    def col_body(g, carry):
        for u in range(4):
            v = g * 4 + u
            cols = jnp.full((16,), v, jnp.int32)
            for e in range(4):
                vals = plsc.load_gather(in_v, [rows[e], cols])
                out_v[pl.ds(v * EMB + e * 16, 16)] = vals
        return carry

    lax.fori_loop(0, ncols // 4, col_body, 0)


def _repack_body(tt_hbm, tail_hbm, out_hbm, in0, in1, of0, of1,
                 sem_in, sem_out):
    wid = lax.axis_index("s") * NC + lax.axis_index("c")
    t0 = wid * MAIN_T
    ins = (in0, in1)
    ofs = (of0, of1)

    def src_at(t):
        return tt_hbm.at[:, pl.ds(pl.multiple_of((t0 + t) * 128, 128), 128)]

    def dst_at(t):
        return out_hbm.at[
            pl.ds(pl.multiple_of((t0 + t) * 128 * EMB, 8), 128 * EMB)]

    pltpu.async_copy(src_at(0), ins[0], sem_in)
    pltpu.async_copy(src_at(1), ins[1], sem_in)

    def body2(t2, carry):
        for b in range(2):
            t = 2 * t2 + b
            pltpu.make_async_copy(src_at(t), ins[b], sem_in).wait()
            @pl.when(t >= 2)
            def _():
                pltpu.make_async_copy(ofs[b], dst_at(t), sem_out).wait()
            _transpose_tile(ins[b], ofs[b], 128)
            pltpu.async_copy(ofs[b], dst_at(t), sem_out)
            @pl.when(t + 2 < MAIN_T)
            def _():
                pltpu.async_copy(src_at(t + 2), ins[b], sem_in)
        return carry

    lax.fori_loop(0, MAIN_T // 2, body2, 0)
    pltpu.make_async_copy(ofs[0], dst_at(0), sem_out).wait()
    pltpu.make_async_copy(ofs[1], dst_at(1), sem_out).wait()

    for k in range(EXTRA):
        @pl.when(wid == k)
        def _():
            col0 = (FULL_TILES - EXTRA + k) * 128
            pltpu.async_copy(tt_hbm.at[:, pl.ds(col0, 128)], in0,
                             sem_in).wait()
            _transpose_tile(in0, of0, 128)
            pltpu.async_copy(of0,
                             out_hbm.at[pl.ds(col0 * EMB, 128 * EMB)],
                             sem_out).wait()

    @pl.when(wid == NW - 1)
    def _():
        col0 = FULL_TILES * 128
        ntail = VOCAB - col0
        pltpu.async_copy(tail_hbm, in0, sem_in).wait()
        _transpose_tile(in0, of0, ntail)
        pltpu.async_copy(of0.at[pl.ds(0, ntail * EMB)],
                         out_hbm.at[pl.ds(col0 * EMB, ntail * EMB)],
                         sem_out).wait()


def _gather_body(idx_hbm, table_hbm, out_hbm, idx_v, buf, out_v, sem):
    wid = lax.axis_index("s") * NC + lax.axis_index("c")
    base = wid * ROWS_PER_W
    pltpu.sync_copy(idx_hbm.at[pl.ds(base * L, ROWS_PER_W * L)], idx_v)

    def accum_body(r, accs):
        new = list(accs)
        for r2 in range(4):
            for c in range(4):
                new[c] = new[c] + buf[r * 4 + r2, pl.ds(c * 16, 16)]
        return tuple(new)

    def row_body(i, carry):
        off = pl.multiple_of(i * L, 8)
        copies = [
            pltpu.async_copy(
                table_hbm.at[idx_v.at[pl.ds(off + c0, n)]],
                buf.at[pl.ds(c0, n)],
                sem,
            )
            for (c0, n) in CHUNKS
        ]
        for c in copies:
            c.wait()
        zeros = tuple(jnp.zeros((16,), jnp.float32) for _ in range(4))
        accs = lax.fori_loop(0, L // 4, accum_body, zeros)
        for c in range(4):
            out_v[i, pl.ds(c * 16, 16)] = accs[c] * (1.0 / L)
        return carry

    lax.fori_loop(0, ROWS_PER_W, row_body, 0)
    pltpu.sync_copy(out_v, out_hbm.at[pl.ds(base, ROWS_PER_W)])


@jax.jit
def _sc_pooled(idx_flat, table):
    mesh = plsc.VectorSubcoreMesh(
        core_axis_name="c", subcore_axis_name="s", num_cores=NC, num_subcores=NS
    )
    tail = jnp.pad(table.T[:, FULL_TILES * 128:], ((0, 0), (0, 64)))
    flat = pl.kernel(
        _repack_body,
        out_type=jax.ShapeDtypeStruct((VOCAB * EMB,), jnp.float32),
        mesh=mesh,
        scratch_types=[
            pltpu.VMEM((EMB, 128), jnp.float32),
            pltpu.VMEM((EMB, 128), jnp.float32),
            pltpu.VMEM((128 * EMB,), jnp.float32),
            pltpu.VMEM((128 * EMB,), jnp.float32),
            pltpu.SemaphoreType.DMA,
            pltpu.SemaphoreType.DMA,
        ],
        compiler_params=pltpu.CompilerParams(
            use_tc_tiling_on_sc=True, needs_layout_passes=False
        ),
        name="sc_repack",
    )(table.T, tail)
    return pl.kernel(
        _gather_body,
        out_type=jax.ShapeDtypeStruct((ROWS_TOTAL, EMB), jnp.float32),
        mesh=mesh,
        scratch_types=[
            pltpu.VMEM((ROWS_PER_W * L,), jnp.int32),
            pltpu.VMEM((L, EMB), jnp.float32),
            pltpu.VMEM((ROWS_PER_W, EMB), jnp.float32),
            pltpu.SemaphoreType.DMA,
        ],
        compiler_params=pltpu.CompilerParams(use_tc_tiling_on_sc=False),
        name="sc_gather_mean",
    )(idx_flat, flat.reshape(VOCAB, EMB))


def _mlp_body(xc_ref, xs_ref, w1_ref, b1_ref, w2_ref, b2_ref, o_ref):
    w1 = w1_ref[...]
    b1 = b1_ref[...]
    w2 = w2_ref[...]
    b2 = b2_ref[...]
    hc = jax.nn.relu(
        jnp.dot(xc_ref[...], w1, preferred_element_type=jnp.float32) + b1
    )
    hs = jax.nn.relu(
        jnp.dot(xs_ref[...], w1, preferred_element_type=jnp.float32) + b1
    )
    hc = jax.nn.relu(jnp.dot(hc, w2, preferred_element_type=jnp.float32) + b2)
    hs = jax.nn.relu(jnp.dot(hs, w2, preferred_element_type=jnp.float32) + b2)
    d = hc - hs
    o_ref[...] = jnp.sqrt(jnp.sum(d * d, axis=1))


@jax.jit
def _mlp_distance(xc, xs, w1t, b1, w2t, b2):
    return pl.pallas_call(
        _mlp_body,
        out_shape=jax.ShapeDtypeStruct((B,), jnp.float32),
    )(xc, xs, w1t, b1, w2t, b2)


def kernel(correct_sql, student_sql, table, fc_w, fc_b, out_w, out_b):
    idx_flat = jnp.concatenate(
        [correct_sql.astype(jnp.int32), student_sql.astype(jnp.int32)], axis=0
    ).reshape(-1)
    pooled = _sc_pooled(idx_flat, table)
    xc = pooled[:B]
    xs = pooled[B:]
    return _mlp_distance(
        xc, xs, fc_w.T, fc_b[None, :], out_w.T, out_b[None, :]
    )


# speedup vs baseline: 2.3882x; 2.0289x over previous
import jax
import jax.numpy as jnp
from jax import lax
from jax.experimental import pallas as pl
from jax.experimental.pallas import tpu as pltpu
from jax.experimental.pallas import tpu_sc as plsc

VOCAB = 1000000
EMB = 64
HID = 128
B = 4096
L = 200

NC = 2
NS = 16
NW = NC * NS
ROWS_TOTAL = 2 * B
ROWS_PER_W = ROWS_TOTAL // NW
FULL_TILES = VOCAB // 128
MAIN_T = FULL_TILES // NW
EXTRA = FULL_TILES - MAIN_T * NW
CHUNKS = ((0, 128), (128, 72))


def _transpose_tile(in_v, out_v, ncols):
    iota = lax.iota(jnp.int32, 16)
    tmod = [(iota + d) & 15 for d in range(16)]
    sbase = [tmod[d] * EMB + iota for d in range(16)]

    def sub_body(s, carry):
        rblk = s & 3
        vblk = s >> 2
        r0 = rblk * 16
        v0 = vblk * 16
        rows = jnp.full((16,), r0, jnp.int32) + iota
        vsplat = jnp.full((16,), v0, jnp.int32)
        osplat = jnp.full((16,), v0 * EMB + r0, jnp.int32)
        for d in range(16):
            cols = vsplat + tmod[d]
            vals = plsc.load_gather(in_v, [rows, cols])
            plsc.store_scatter(out_v, [osplat + sbase[d]], vals)
        return carry

    lax.fori_loop(0, 4 * (ncols // 16), sub_body, 0)


def _repack_body(tt_hbm, tail_hbm, out_hbm, in0, in1, of0, of1,
                 sem_in, sem_out):
    wid = lax.axis_index("s") * NC + lax.axis_index("c")
    t0 = wid * MAIN_T
    ins = (in0, in1)
    ofs = (of0, of1)

    def src_at(t):
        return tt_hbm.at[:, pl.ds(pl.multiple_of((t0 + t) * 128, 128), 128)]

    def dst_at(t):
        return out_hbm.at[
            pl.ds(pl.multiple_of((t0 + t) * 128 * EMB, 8), 128 * EMB)]

    pltpu.async_copy(src_at(0), ins[0], sem_in)
    pltpu.async_copy(src_at(1), ins[1], sem_in)

    def body2(t2, carry):
        for b in range(2):
            t = 2 * t2 + b
            pltpu.make_async_copy(src_at(t), ins[b], sem_in).wait()
            @pl.when(t >= 2)
            def _():
                pltpu.make_async_copy(ofs[b], dst_at(t), sem_out).wait()
            _transpose_tile(ins[b], ofs[b], 128)
            pltpu.async_copy(ofs[b], dst_at(t), sem_out)
            @pl.when(t + 2 < MAIN_T)
            def _():
                pltpu.async_copy(src_at(t + 2), ins[b], sem_in)
        return carry

    lax.fori_loop(0, MAIN_T // 2, body2, 0)
    pltpu.make_async_copy(ofs[0], dst_at(0), sem_out).wait()
    pltpu.make_async_copy(ofs[1], dst_at(1), sem_out).wait()

    for k in range(EXTRA):
        @pl.when(wid == k)
        def _():
            col0 = (FULL_TILES - EXTRA + k) * 128
            pltpu.async_copy(tt_hbm.at[:, pl.ds(col0, 128)], in0,
                             sem_in).wait()
            _transpose_tile(in0, of0, 128)
            pltpu.async_copy(of0,
                             out_hbm.at[pl.ds(col0 * EMB, 128 * EMB)],
                             sem_out).wait()

    @pl.when(wid == NW - 1)
    def _():
        col0 = FULL_TILES * 128
        ntail = VOCAB - col0
        pltpu.async_copy(tail_hbm, in0, sem_in).wait()
        _transpose_tile(in0, of0, ntail)
        pltpu.async_copy(of0.at[pl.ds(0, ntail * EMB)],
                         out_hbm.at[pl.ds(col0 * EMB, ntail * EMB)],
                         sem_out).wait()


def _gather_body(idx_hbm, table_hbm, out_hbm, idx_v, buf, out_v, sem):
    wid = lax.axis_index("s") * NC + lax.axis_index("c")
    base = wid * ROWS_PER_W
    pltpu.sync_copy(idx_hbm.at[pl.ds(base * L, ROWS_PER_W * L)], idx_v)

    def accum_body(r, accs):
        new = list(accs)
        for r2 in range(4):
            for c in range(4):
                new[c] = new[c] + buf[r * 4 + r2, pl.ds(c * 16, 16)]
        return tuple(new)

    def row_body(i, carry):
        off = pl.multiple_of(i * L, 8)
        copies = [
            pltpu.async_copy(
                table_hbm.at[idx_v.at[pl.ds(off + c0, n)]],
                buf.at[pl.ds(c0, n)],
                sem,
            )
            for (c0, n) in CHUNKS
        ]
        for c in copies:
            c.wait()
        zeros = tuple(jnp.zeros((16,), jnp.float32) for _ in range(4))
        accs = lax.fori_loop(0, L // 4, accum_body, zeros)
        for c in range(4):
            out_v[i, pl.ds(c * 16, 16)] = accs[c] * (1.0 / L)
        return carry

    lax.fori_loop(0, ROWS_PER_W, row_body, 0)
    pltpu.sync_copy(out_v, out_hbm.at[pl.ds(base, ROWS_PER_W)])


@jax.jit
def _sc_pooled(idx_flat, table):
    mesh = plsc.VectorSubcoreMesh(
        core_axis_name="c", subcore_axis_name="s", num_cores=NC, num_subcores=NS
    )
    tail = jnp.pad(table.T[:, FULL_TILES * 128:], ((0, 0), (0, 64)))
    flat = pl.kernel(
        _repack_body,
        out_type=jax.ShapeDtypeStruct((VOCAB * EMB,), jnp.float32),
        mesh=mesh,
        scratch_types=[
            pltpu.VMEM((EMB, 128), jnp.float32),
            pltpu.VMEM((EMB, 128), jnp.float32),
            pltpu.VMEM((128 * EMB,), jnp.float32),
            pltpu.VMEM((128 * EMB,), jnp.float32),
            pltpu.SemaphoreType.DMA,
            pltpu.SemaphoreType.DMA,
        ],
        compiler_params=pltpu.CompilerParams(
            use_tc_tiling_on_sc=True, needs_layout_passes=False
        ),
        name="sc_repack",
    )(table.T, tail)
    return pl.kernel(
        _gather_body,
        out_type=jax.ShapeDtypeStruct((ROWS_TOTAL, EMB), jnp.float32),
        mesh=mesh,
        scratch_types=[
            pltpu.VMEM((ROWS_PER_W * L,), jnp.int32),
            pltpu.VMEM((L, EMB), jnp.float32),
            pltpu.VMEM((ROWS_PER_W, EMB), jnp.float32),
            pltpu.SemaphoreType.DMA,
        ],
        compiler_params=pltpu.CompilerParams(use_tc_tiling_on_sc=False),
        name="sc_gather_mean",
    )(idx_flat, flat.reshape(VOCAB, EMB))


def _mlp_body(xc_ref, xs_ref, w1_ref, b1_ref, w2_ref, b2_ref, o_ref):
    w1 = w1_ref[...]
    b1 = b1_ref[...]
    w2 = w2_ref[...]
    b2 = b2_ref[...]
    hc = jax.nn.relu(
        jnp.dot(xc_ref[...], w1, preferred_element_type=jnp.float32) + b1
    )
    hs = jax.nn.relu(
        jnp.dot(xs_ref[...], w1, preferred_element_type=jnp.float32) + b1
    )
    hc = jax.nn.relu(jnp.dot(hc, w2, preferred_element_type=jnp.float32) + b2)
    hs = jax.nn.relu(jnp.dot(hs, w2, preferred_element_type=jnp.float32) + b2)
    d = hc - hs
    o_ref[...] = jnp.sqrt(jnp.sum(d * d, axis=1))


@jax.jit
def _mlp_distance(xc, xs, w1t, b1, w2t, b2):
    return pl.pallas_call(
        _mlp_body,
        out_shape=jax.ShapeDtypeStruct((B,), jnp.float32),
    )(xc, xs, w1t, b1, w2t, b2)


def kernel(correct_sql, student_sql, table, fc_w, fc_b, out_w, out_b):
    idx_flat = jnp.concatenate(
        [correct_sql.astype(jnp.int32), student_sql.astype(jnp.int32)], axis=0
    ).reshape(-1)
    pooled = _sc_pooled(idx_flat, table)
    xc = pooled[:B]
    xs = pooled[B:]
    return _mlp_distance(
        xc, xs, fc_w.T, fc_b[None, :], out_w.T, out_b[None, :]
    )


# speedup vs baseline: 2.7090x; 1.1343x over previous
import jax
import jax.numpy as jnp
from jax import lax
from jax.experimental import pallas as pl
from jax.experimental.pallas import tpu as pltpu
from jax.experimental.pallas import tpu_sc as plsc

VOCAB = 1000000
EMB = 64
HID = 128
B = 4096
L = 200

NC = 2
NS = 16
NW = NC * NS
ROWS_TOTAL = 2 * B
ROWS_PER_W = ROWS_TOTAL // NW
FULL_TILES = VOCAB // 128
MAIN_T = FULL_TILES // NW
EXTRA = FULL_TILES - MAIN_T * NW
CHUNKS = ((0, 128), (128, 72))


def _transpose_tile(in_v, out_v, ncols):
    iota = lax.iota(jnp.int32, 16)
    tmod = [(iota + d) & 15 for d in range(16)]
    sbase = [tmod[d] * EMB + iota for d in range(16)]

    def sub_body(s, carry):
        rblk = s & 3
        vblk = s >> 2
        r0 = rblk * 16
        v0 = vblk * 16
        rows = jnp.full((16,), r0, jnp.int32) + iota
        vsplat = jnp.full((16,), v0, jnp.int32)
        osplat = jnp.full((16,), v0 * EMB + r0, jnp.int32)
        for d in range(16):
            cols = vsplat + tmod[d]
            vals = plsc.load_gather(in_v, [rows, cols])
            plsc.store_scatter(out_v, [osplat + sbase[d]], vals)
        return carry

    lax.fori_loop(0, 4 * (ncols // 16), sub_body, 0)


def _repack_body(tt_hbm, tail_hbm, out_hbm, in0, in1, of0, of1,
                 sem_in, sem_out):
    wid = lax.axis_index("s") * NC + lax.axis_index("c")
    t0 = wid * MAIN_T
    ins = (in0, in1)
    ofs = (of0, of1)

    def src_at(t):
        return tt_hbm.at[:, pl.ds(pl.multiple_of((t0 + t) * 128, 128), 128)]

    def dst_at(t):
        return out_hbm.at[
            pl.ds(pl.multiple_of((t0 + t) * 128 * EMB, 8), 128 * EMB)]

    pltpu.async_copy(src_at(0), ins[0], sem_in)
    pltpu.async_copy(src_at(1), ins[1], sem_in)

    def body2(t2, carry):
        for b in range(2):
            t = 2 * t2 + b
            pltpu.make_async_copy(src_at(t), ins[b], sem_in).wait()
            @pl.when(t >= 2)
            def _():
                pltpu.make_async_copy(ofs[b], dst_at(t), sem_out).wait()
            _transpose_tile(ins[b], ofs[b], 128)
            pltpu.async_copy(ofs[b], dst_at(t), sem_out)
            @pl.when(t + 2 < MAIN_T)
            def _():
                pltpu.async_copy(src_at(t + 2), ins[b], sem_in)
        return carry

    lax.fori_loop(0, MAIN_T // 2, body2, 0)
    pltpu.make_async_copy(ofs[0], dst_at(0), sem_out).wait()
    pltpu.make_async_copy(ofs[1], dst_at(1), sem_out).wait()

    for k in range(EXTRA):
        @pl.when(wid == k)
        def _():
            col0 = (FULL_TILES - EXTRA + k) * 128
            pltpu.async_copy(tt_hbm.at[:, pl.ds(col0, 128)], in0,
                             sem_in).wait()
            _transpose_tile(in0, of0, 128)
            pltpu.async_copy(of0,
                             out_hbm.at[pl.ds(col0 * EMB, 128 * EMB)],
                             sem_out).wait()

    @pl.when(wid == NW - 1)
    def _():
        col0 = FULL_TILES * 128
        ntail = VOCAB - col0
        pltpu.async_copy(tail_hbm, in0, sem_in).wait()
        _transpose_tile(in0, of0, ntail)
        pltpu.async_copy(of0.at[pl.ds(0, ntail * EMB)],
                         out_hbm.at[pl.ds(col0 * EMB, ntail * EMB)],
                         sem_out).wait()


def _gather_body(idx_hbm, table_hbm, out_hbm, idx_v, buf0, buf1, out_v, sem):
    wid = lax.axis_index("s") * NC + lax.axis_index("c")
    base = wid * ROWS_PER_W
    pltpu.sync_copy(idx_hbm.at[pl.ds(base * L, ROWS_PER_W * L)], idx_v)
    bufs = (buf0, buf1)

    def gathers(i, buf):
        off = pl.multiple_of(i * L, 8)
        return [
            pltpu.make_async_copy(
                table_hbm.at[idx_v.at[pl.ds(off + c0, n)]],
                buf.at[pl.ds(c0, n)],
                sem,
            )
            for (c0, n) in CHUNKS
        ]

    def make_accum(buf):
        def accum_body(r, accs):
            new = list(accs)
            for r2 in range(4):
                for c in range(4):
                    new[c] = new[c] + buf[r * 4 + r2, pl.ds(c * 16, 16)]
            return tuple(new)
        return accum_body

    accum_bodies = (make_accum(buf0), make_accum(buf1))

    for c in gathers(0, bufs[0]):
        c.start()

    def row2_body(t, carry):
        for b in range(2):
            i = 2 * t + b
            for c in gathers(i, bufs[b]):
                c.wait()

            @pl.when(i + 1 < ROWS_PER_W)
            def _():
                for c in gathers(i + 1, bufs[1 - b]):
                    c.start()

            zeros = tuple(jnp.zeros((16,), jnp.float32) for _ in range(4))
            accs = lax.fori_loop(0, L // 4, accum_bodies[b], zeros)
            for c in range(4):
                out_v[i, pl.ds(c * 16, 16)] = accs[c] * (1.0 / L)
        return carry

    lax.fori_loop(0, ROWS_PER_W // 2, row2_body, 0)
    pltpu.sync_copy(out_v, out_hbm.at[pl.ds(base, ROWS_PER_W)])


@jax.jit
def _sc_pooled(idx_flat, table):
    mesh = plsc.VectorSubcoreMesh(
        core_axis_name="c", subcore_axis_name="s", num_cores=NC, num_subcores=NS
    )
    tail = jnp.pad(table.T[:, FULL_TILES * 128:], ((0, 0), (0, 64)))
    flat = pl.kernel(
        _repack_body,
        out_type=jax.ShapeDtypeStruct((VOCAB * EMB,), jnp.float32),
        mesh=mesh,
        scratch_types=[
            pltpu.VMEM((EMB, 128), jnp.float32),
            pltpu.VMEM((EMB, 128), jnp.float32),
            pltpu.VMEM((128 * EMB,), jnp.float32),
            pltpu.VMEM((128 * EMB,), jnp.float32),
            pltpu.SemaphoreType.DMA,
            pltpu.SemaphoreType.DMA,
        ],
        compiler_params=pltpu.CompilerParams(
            use_tc_tiling_on_sc=True, needs_layout_passes=False
        ),
        name="sc_repack",
    )(table.T, tail)
    return pl.kernel(
        _gather_body,
        out_type=jax.ShapeDtypeStruct((ROWS_TOTAL, EMB), jnp.float32),
        mesh=mesh,
        scratch_types=[
            pltpu.VMEM((ROWS_PER_W * L,), jnp.int32),
            pltpu.VMEM((L, EMB), jnp.float32),
            pltpu.VMEM((L, EMB), jnp.float32),
            pltpu.VMEM((ROWS_PER_W, EMB), jnp.float32),
            pltpu.SemaphoreType.DMA,
        ],
        compiler_params=pltpu.CompilerParams(use_tc_tiling_on_sc=False),
        name="sc_gather_mean",
    )(idx_flat, flat.reshape(VOCAB, EMB))


def _mlp_body(xc_ref, xs_ref, w1_ref, b1_ref, w2_ref, b2_ref, o_ref):
    w1 = w1_ref[...]
    b1 = b1_ref[...]
    w2 = w2_ref[...]
    b2 = b2_ref[...]
    hc = jax.nn.relu(
        jnp.dot(xc_ref[...], w1, preferred_element_type=jnp.float32) + b1
    )
    hs = jax.nn.relu(
        jnp.dot(xs_ref[...], w1, preferred_element_type=jnp.float32) + b1
    )
    hc = jax.nn.relu(jnp.dot(hc, w2, preferred_element_type=jnp.float32) + b2)
    hs = jax.nn.relu(jnp.dot(hs, w2, preferred_element_type=jnp.float32) + b2)
    d = hc - hs
    o_ref[...] = jnp.sqrt(jnp.sum(d * d, axis=1))


@jax.jit
def _mlp_distance(xc, xs, w1t, b1, w2t, b2):
    return pl.pallas_call(
        _mlp_body,
        out_shape=jax.ShapeDtypeStruct((B,), jnp.float32),
    )(xc, xs, w1t, b1, w2t, b2)


def kernel(correct_sql, student_sql, table, fc_w, fc_b, out_w, out_b):
    idx_flat = jnp.concatenate(
        [correct_sql.astype(jnp.int32), student_sql.astype(jnp.int32)], axis=0
    ).reshape(-1)
    pooled = _sc_pooled(idx_flat, table)
    xc = pooled[:B]
    xs = pooled[B:]
    return _mlp_distance(
        xc, xs, fc_w.T, fc_b[None, :], out_w.T, out_b[None, :]
    )


# speedup vs baseline: 2.7130x; 1.0015x over previous
import jax
import jax.numpy as jnp
from jax import lax
from jax.experimental import pallas as pl
from jax.experimental.pallas import tpu as pltpu
from jax.experimental.pallas import tpu_sc as plsc

VOCAB = 1000000
EMB = 64
HID = 128
B = 4096
L = 200

NC = 2
NS = 16
NW = NC * NS
ROWS_TOTAL = 2 * B
ROWS_PER_W = ROWS_TOTAL // NW
FULL_TILES = VOCAB // 128
MAIN_T = FULL_TILES // NW
EXTRA = FULL_TILES - MAIN_T * NW
CHUNKS = ((0, 128), (128, 72))


def _transpose_tile(in_v, out_v, ncols):
    iota = lax.iota(jnp.int32, 16)
    tmod = [(iota + d) & 15 for d in range(16)]
    sbase = [tmod[d] * EMB + iota for d in range(16)]

    def sub_body(s2, carry):
        for u in range(2):
            s = s2 * 2 + u
            rblk = s & 3
            vblk = s >> 2
            r0 = rblk * 16
            v0 = vblk * 16
            rows = jnp.full((16,), r0, jnp.int32) + iota
            vsplat = jnp.full((16,), v0, jnp.int32)
            osplat = jnp.full((16,), v0 * EMB + r0, jnp.int32)
            for d in range(16):
                cols = vsplat + tmod[d]
                vals = plsc.load_gather(in_v, [rows, cols])
                plsc.store_scatter(out_v, [osplat + sbase[d]], vals)
        return carry

    lax.fori_loop(0, 2 * (ncols // 16), sub_body, 0)


def _repack_body(tt_hbm, tail_hbm, out_hbm, in0, in1, of0, of1,
                 sem_in, sem_out):
    wid = lax.axis_index("s") * NC + lax.axis_index("c")
    t0 = wid * MAIN_T
    ins = (in0, in1)
    ofs = (of0, of1)

    def src_at(t):
        return tt_hbm.at[:, pl.ds(pl.multiple_of((t0 + t) * 128, 128), 128)]

    def dst_at(t):
        return out_hbm.at[
            pl.ds(pl.multiple_of((t0 + t) * 128 * EMB, 8), 128 * EMB)]

    pltpu.async_copy(src_at(0), ins[0], sem_in)
    pltpu.async_copy(src_at(1), ins[1], sem_in)

    def body2(t2, carry):
        for b in range(2):
            t = 2 * t2 + b
            pltpu.make_async_copy(src_at(t), ins[b], sem_in).wait()
            @pl.when(t >= 2)
            def _():
                pltpu.make_async_copy(ofs[b], dst_at(t), sem_out).wait()
            _transpose_tile(ins[b], ofs[b], 128)
            pltpu.async_copy(ofs[b], dst_at(t), sem_out)
            @pl.when(t + 2 < MAIN_T)
            def _():
                pltpu.async_copy(src_at(t + 2), ins[b], sem_in)
        return carry

    lax.fori_loop(0, MAIN_T // 2, body2, 0)
    pltpu.make_async_copy(ofs[0], dst_at(0), sem_out).wait()
    pltpu.make_async_copy(ofs[1], dst_at(1), sem_out).wait()

    for k in range(EXTRA):
        @pl.when(wid == k)
        def _():
            col0 = (FULL_TILES - EXTRA + k) * 128
            pltpu.async_copy(tt_hbm.at[:, pl.ds(col0, 128)], in0,
                             sem_in).wait()
            _transpose_tile(in0, of0, 128)
            pltpu.async_copy(of0,
                             out_hbm.at[pl.ds(col0 * EMB, 128 * EMB)],
                             sem_out).wait()

    @pl.when(wid == NW - 1)
    def _():
        col0 = FULL_TILES * 128
        ntail = VOCAB - col0
        pltpu.async_copy(tail_hbm, in0, sem_in).wait()
        _transpose_tile(in0, of0, ntail)
        pltpu.async_copy(of0.at[pl.ds(0, ntail * EMB)],
                         out_hbm.at[pl.ds(col0 * EMB, ntail * EMB)],
                         sem_out).wait()


def _gather_body(idx_hbm, table_hbm, out_hbm, idx_v, buf0, buf1, out_v, sem):
    wid = lax.axis_index("s") * NC + lax.axis_index("c")
    base = wid * ROWS_PER_W
    pltpu.sync_copy(idx_hbm.at[pl.ds(base * L, ROWS_PER_W * L)], idx_v)
    bufs = (buf0, buf1)

    def gathers(i, buf):
        off = pl.multiple_of(i * L, 8)
        return [
            pltpu.make_async_copy(
                table_hbm.at[idx_v.at[pl.ds(off + c0, n)]],
                buf.at[pl.ds(c0, n)],
                sem,
            )
            for (c0, n) in CHUNKS
        ]

    def make_accum(buf):
        def accum_body(r, accs):
            new = list(accs)
            for r2 in range(4):
                for c in range(4):
                    new[c] = new[c] + buf[r * 4 + r2, pl.ds(c * 16, 16)]
            return tuple(new)
        return accum_body

    accum_bodies = (make_accum(buf0), make_accum(buf1))

    for c in gathers(0, bufs[0]):
        c.start()

    def row2_body(t, carry):
        for b in range(2):
            i = 2 * t + b
            for c in gathers(i, bufs[b]):
                c.wait()

            @pl.when(i + 1 < ROWS_PER_W)
            def _():
                for c in gathers(i + 1, bufs[1 - b]):
                    c.start()

            zeros = tuple(jnp.zeros((16,), jnp.float32) for _ in range(4))
            accs = lax.fori_loop(0, L // 4, accum_bodies[b], zeros)
            for c in range(4):
                out_v[i, pl.ds(c * 16, 16)] = accs[c] * (1.0 / L)
        return carry

    lax.fori_loop(0, ROWS_PER_W // 2, row2_body, 0)
    pltpu.sync_copy(out_v, out_hbm.at[pl.ds(base, ROWS_PER_W)])


@jax.jit
def _sc_pooled(idx_flat, table):
    mesh = plsc.VectorSubcoreMesh(
        core_axis_name="c", subcore_axis_name="s", num_cores=NC, num_subcores=NS
    )
    tail = jnp.pad(table.T[:, FULL_TILES * 128:], ((0, 0), (0, 64)))
    flat = pl.kernel(
        _repack_body,
        out_type=jax.ShapeDtypeStruct((VOCAB * EMB,), jnp.float32),
        mesh=mesh,
        scratch_types=[
            pltpu.VMEM((EMB, 128), jnp.float32),
            pltpu.VMEM((EMB, 128), jnp.float32),
            pltpu.VMEM((128 * EMB,), jnp.float32),
            pltpu.VMEM((128 * EMB,), jnp.float32),
            pltpu.SemaphoreType.DMA,
            pltpu.SemaphoreType.DMA,
        ],
        compiler_params=pltpu.CompilerParams(
            use_tc_tiling_on_sc=True, needs_layout_passes=False
        ),
        name="sc_repack",
    )(table.T, tail)
    return pl.kernel(
        _gather_body,
        out_type=jax.ShapeDtypeStruct((ROWS_TOTAL, EMB), jnp.float32),
        mesh=mesh,
        scratch_types=[
            pltpu.VMEM((ROWS_PER_W * L,), jnp.int32),
            pltpu.VMEM((L, EMB), jnp.float32),
            pltpu.VMEM((L, EMB), jnp.float32),
            pltpu.VMEM((ROWS_PER_W, EMB), jnp.float32),
            pltpu.SemaphoreType.DMA,
        ],
        compiler_params=pltpu.CompilerParams(use_tc_tiling_on_sc=False),
        name="sc_gather_mean",
    )(idx_flat, flat.reshape(VOCAB, EMB))


def _mlp_body(xc_ref, xs_ref, w1_ref, b1_ref, w2_ref, b2_ref, o_ref):
    w1 = w1_ref[...]
    b1 = b1_ref[...]
    w2 = w2_ref[...]
    b2 = b2_ref[...]
    hc = jax.nn.relu(
        jnp.dot(xc_ref[...], w1, preferred_element_type=jnp.float32) + b1
    )
    hs = jax.nn.relu(
        jnp.dot(xs_ref[...], w1, preferred_element_type=jnp.float32) + b1
    )
    hc = jax.nn.relu(jnp.dot(hc, w2, preferred_element_type=jnp.float32) + b2)
    hs = jax.nn.relu(jnp.dot(hs, w2, preferred_element_type=jnp.float32) + b2)
    d = hc - hs
    o_ref[...] = jnp.sqrt(jnp.sum(d * d, axis=1))


@jax.jit
def _mlp_distance(xc, xs, w1t, b1, w2t, b2):
    return pl.pallas_call(
        _mlp_body,
        out_shape=jax.ShapeDtypeStruct((B,), jnp.float32),
    )(xc, xs, w1t, b1, w2t, b2)


def kernel(correct_sql, student_sql, table, fc_w, fc_b, out_w, out_b):
    idx_flat = jnp.concatenate(
        [correct_sql.astype(jnp.int32), student_sql.astype(jnp.int32)], axis=0
    ).reshape(-1)
    pooled = _sc_pooled(idx_flat, table)
    xc = pooled[:B]
    xs = pooled[B:]
    return _mlp_distance(
        xc, xs, fc_w.T, fc_b[None, :], out_w.T, out_b[None, :]
    )
